# baseline (device time: 220942 ns/iter reference)
import jax
import jax.numpy as jnp
from jax import lax
from jax.experimental import pallas as pl
from jax.experimental.pallas import tpu as pltpu

B, SQ, SKV, H, D = 4, 32, 4096, 8, 128
HD = H * D
SCALE = D ** -0.5

BKV = 1024
NKV = SKV // BKV


def _flash_body(q_ref, k_ref, v_ref, o_ref, m_ref, l_ref, acc_o, acc_m, acc_l):
    nk = pl.program_id(1)

    @pl.when(nk == 0)
    def _():
        acc_m[...] = jnp.full((H, SQ, 1), -jnp.inf, jnp.float32)
        acc_l[...] = jnp.zeros((H, SQ, 1), jnp.float32)
        acc_o[...] = jnp.zeros((SQ, HD), jnp.float32)

    for h in range(H):
        sl = slice(h * D, (h + 1) * D)
        q = q_ref[0, :, sl]
        k = k_ref[0, :, sl]
        v = v_ref[0, :, sl]
        s = lax.dot_general(
            q, k, (((1,), (1,)), ((), ())), preferred_element_type=jnp.float32
        ) * SCALE
        m_prev = acc_m[h]
        m_blk = jnp.max(s, axis=1, keepdims=True)
        m_new = jnp.maximum(m_prev, m_blk)
        alpha = jnp.exp(m_prev - m_new)
        p = jnp.exp(s - m_new)
        l_new = acc_l[h] * alpha + jnp.sum(p, axis=1, keepdims=True)
        o_new = acc_o[:, sl] * alpha + lax.dot_general(
            p, v, (((1,), (0,)), ((), ())), preferred_element_type=jnp.float32
        )
        acc_m[h] = m_new
        acc_l[h] = l_new
        acc_o[:, sl] = o_new

    @pl.when(nk == NKV - 1)
    def _():
        for h in range(H):
            sl = slice(h * D, (h + 1) * D)
            o_ref[0, :, sl] = acc_o[:, sl] / acc_l[h]
            m_ref[0, h] = acc_m[h]
            l_ref[0, h] = acc_l[h]


def _local_flash(Q, K, V):
    return pl.pallas_call(
        _flash_body,
        grid=(B, NKV),
        in_specs=[
            pl.BlockSpec((1, SQ, HD), lambda b, nk: (b, 0, 0)),
            pl.BlockSpec((1, BKV, HD), lambda b, nk: (b, nk, 0)),
            pl.BlockSpec((1, BKV, HD), lambda b, nk: (b, nk, 0)),
        ],
        out_shape=[
            jax.ShapeDtypeStruct((B, SQ, HD), jnp.float32),
            jax.ShapeDtypeStruct((B, H, SQ, 1), jnp.float32),
            jax.ShapeDtypeStruct((B, H, SQ, 1), jnp.float32),
        ],
        out_specs=[
            pl.BlockSpec((1, SQ, HD), lambda b, nk: (b, 0, 0)),
            pl.BlockSpec((1, H, SQ, 1), lambda b, nk: (b, 0, 0, 0)),
            pl.BlockSpec((1, H, SQ, 1), lambda b, nk: (b, 0, 0, 0)),
        ],
        scratch_shapes=[
            pltpu.VMEM((SQ, HD), jnp.float32),
            pltpu.VMEM((H, SQ, 1), jnp.float32),
            pltpu.VMEM((H, SQ, 1), jnp.float32),
        ],
    )(Q, K, V)


def _combine_body(
    o_ref, m_ref, l_ref, out_ref, ro_ref, rm_ref, rl_ref, send_sems, recv_sems
):
    x = lax.axis_index("x")
    y = lax.axis_index("y")
    z = lax.axis_index("z")
    partner = (x, y, 1 - z)

    barrier = pltpu.get_barrier_semaphore()
    pl.semaphore_signal(
        barrier, inc=1, device_id=partner, device_id_type=pl.DeviceIdType.MESH
    )
    pl.semaphore_wait(barrier, 1)

    copies = []
    for i, (src, dst) in enumerate(
        ((o_ref, ro_ref), (m_ref, rm_ref), (l_ref, rl_ref))
    ):
        c = pltpu.make_async_remote_copy(
            src_ref=src,
            dst_ref=dst,
            send_sem=send_sems.at[i],
            recv_sem=recv_sems.at[i],
            device_id=partner,
            device_id_type=pl.DeviceIdType.MESH,
        )
        c.start()
        copies.append(c)
    for c in copies:
        c.wait()

    m_a = m_ref[...]
    m_b = rm_ref[...]
    m_n = jnp.maximum(m_a, m_b)
    a = jnp.exp(m_a - m_n) * l_ref[...]
    b = jnp.exp(m_b - m_n) * rl_ref[...]
    denom = a + b
    wa = a / denom
    wb = b / denom
    for bi in range(B):
        for h in range(H):
            sl = slice(h * D, (h + 1) * D)
            out_ref[bi, :, sl] = (
                o_ref[bi, :, sl] * wa[bi, h] + ro_ref[bi, :, sl] * wb[bi, h]
            )


def _combine(o_part, m, l):
    return pl.pallas_call(
        _combine_body,
        in_specs=[
            pl.BlockSpec(memory_space=pltpu.VMEM),
            pl.BlockSpec(memory_space=pltpu.VMEM),
            pl.BlockSpec(memory_space=pltpu.VMEM),
        ],
        out_shape=jax.ShapeDtypeStruct((B, SQ, HD), jnp.float32),
        out_specs=pl.BlockSpec(memory_space=pltpu.VMEM),
        scratch_shapes=[
            pltpu.VMEM((B, SQ, HD), jnp.float32),
            pltpu.VMEM((B, H, SQ, 1), jnp.float32),
            pltpu.VMEM((B, H, SQ, 1), jnp.float32),
            pltpu.SemaphoreType.DMA((3,)),
            pltpu.SemaphoreType.DMA((3,)),
        ],
        compiler_params=pltpu.CompilerParams(collective_id=0),
    )(o_part, m, l)


def kernel(Q, K, V):
    Qr = Q.reshape(B, SQ, HD)
    Kr = K.reshape(B, SKV, HD)
    Vr = V.reshape(B, SKV, HD)
    o_part, m, l = _local_flash(Qr, Kr, Vr)
    out = _combine(o_part, m, l)
    return out.reshape(B, SQ, H, D)


# device time: 36494 ns/iter; 6.0542x vs baseline; 6.0542x over previous
import jax
import jax.numpy as jnp
from jax import lax
from jax.experimental import pallas as pl
from jax.experimental.pallas import tpu as pltpu

B, SQ, SKV, H, D = 4, 32, 4096, 8, 128
SCALE = D ** -0.5

BKV = 1024
NKV = SKV // BKV


def _flash_body(
    b_ref, q_ref, k_ref, v_ref, o_ref, m_ref, l_ref, acc_o, acc_m, acc_l, kt, vt
):
    nk = pl.program_id(0)

    @pl.when(nk == 0)
    def _():
        acc_m[...] = jnp.full((H, SQ, 1), -jnp.inf, jnp.float32)
        acc_l[...] = jnp.zeros((H, SQ, 1), jnp.float32)
        acc_o[...] = jnp.zeros((SQ, H, D), jnp.float32)

    kt[...] = jnp.swapaxes(k_ref[0], 0, 1)
    vt[...] = jnp.swapaxes(v_ref[0], 0, 1)
    for h in range(H):
        q = q_ref[0, :, h, :]
        k = kt[h]
        v = vt[h]
        s = lax.dot_general(
            q, k, (((1,), (1,)), ((), ())), preferred_element_type=jnp.float32
        ) * SCALE
        m_prev = acc_m[h]
        m_blk = jnp.max(s, axis=1, keepdims=True)
        m_new = jnp.maximum(m_prev, m_blk)
        alpha = jnp.exp(m_prev - m_new)
        p = jnp.exp(s - m_new)
        l_new = acc_l[h] * alpha + jnp.sum(p, axis=1, keepdims=True)
        o_new = acc_o[:, h, :] * alpha + lax.dot_general(
            p, v, (((1,), (0,)), ((), ())), preferred_element_type=jnp.float32
        )
        acc_m[h] = m_new
        acc_l[h] = l_new
        acc_o[:, h, :] = o_new

    @pl.when(nk == NKV - 1)
    def _():
        for h in range(H):
            o_ref[0, :, h, :] = acc_o[:, h, :] / acc_l[h]
            m_ref[0, h] = acc_m[h]
            l_ref[0, h] = acc_l[h]


def _local_flash(b_idx, Q, K, V):
    return pl.pallas_call(
        _flash_body,
        grid_spec=pltpu.PrefetchScalarGridSpec(
            num_scalar_prefetch=1,
            grid=(NKV,),
            in_specs=[
                pl.BlockSpec((1, SQ, H, D), lambda nk, b: (b[0], 0, 0, 0)),
                pl.BlockSpec((1, BKV, H, D), lambda nk, b: (b[0], nk, 0, 0)),
                pl.BlockSpec((1, BKV, H, D), lambda nk, b: (b[0], nk, 0, 0)),
            ],
            out_specs=[
                pl.BlockSpec((1, SQ, H, D), lambda nk, b: (0, 0, 0, 0)),
                pl.BlockSpec((1, H, SQ, 1), lambda nk, b: (0, 0, 0, 0)),
                pl.BlockSpec((1, H, SQ, 1), lambda nk, b: (0, 0, 0, 0)),
            ],
            scratch_shapes=[
                pltpu.VMEM((SQ, H, D), jnp.float32),
                pltpu.VMEM((H, SQ, 1), jnp.float32),
                pltpu.VMEM((H, SQ, 1), jnp.float32),
                pltpu.VMEM((H, BKV, D), jnp.float32),
                pltpu.VMEM((H, BKV, D), jnp.float32),
            ],
        ),
        out_shape=[
            jax.ShapeDtypeStruct((1, SQ, H, D), jnp.float32),
            jax.ShapeDtypeStruct((1, H, SQ, 1), jnp.float32),
            jax.ShapeDtypeStruct((1, H, SQ, 1), jnp.float32),
        ],
    )(b_idx, Q, K, V)


def _combine_body(
    o_ref,
    m_ref,
    l_ref,
    out_ref,
    ro_ref,
    rm_ref,
    rl_ref,
    comb_ref,
    zsend,
    zrecv,
    psend,
    precv,
):
    x = lax.axis_index("x")
    y = lax.axis_index("y")
    z = lax.axis_index("z")
    my_b = 2 * x + y
    partner = (x, y, 1 - z)
    xn = (1 - x, y, z)
    yn = (x, 1 - y, z)
    dg = (1 - x, 1 - y, z)
    b_xn = 2 * (1 - x) + y
    b_yn = 2 * x + (1 - y)
    b_dg = 2 * (1 - x) + (1 - y)

    barrier = pltpu.get_barrier_semaphore()
    for nbr in (partner, xn, yn, dg):
        pl.semaphore_signal(
            barrier, inc=1, device_id=nbr, device_id_type=pl.DeviceIdType.MESH
        )
    pl.semaphore_wait(barrier, 4)

    zc = []
    for i, (src, dst) in enumerate(
        ((o_ref, ro_ref), (m_ref, rm_ref), (l_ref, rl_ref))
    ):
        c = pltpu.make_async_remote_copy(
            src_ref=src,
            dst_ref=dst,
            send_sem=zsend.at[i],
            recv_sem=zrecv.at[i],
            device_id=partner,
            device_id_type=pl.DeviceIdType.MESH,
        )
        c.start()
        zc.append(c)
    for c in zc:
        c.wait()

    m_a = m_ref[...]
    m_b = rm_ref[...]
    m_n = jnp.maximum(m_a, m_b)
    a = jnp.exp(m_a - m_n) * l_ref[...]
    bb = jnp.exp(m_b - m_n) * rl_ref[...]
    den = a + bb
    wa = a / den
    wb = bb / den
    for h in range(H):
        comb_ref[0, :, h, :] = (
            o_ref[0, :, h, :] * wa[0, h] + ro_ref[0, :, h, :] * wb[0, h]
        )
    out_ref[pl.ds(my_b, 1)] = comb_ref[...]

    sends = []
    for i, tgt in enumerate((xn, yn, dg)):
        c = pltpu.make_async_remote_copy(
            src_ref=comb_ref,
            dst_ref=out_ref.at[pl.ds(my_b, 1)],
            send_sem=psend.at[i],
            recv_sem=precv.at[i],
            device_id=tgt,
            device_id_type=pl.DeviceIdType.MESH,
        )
        c.start()
        sends.append(c)
    recvs = []
    for i, bsrc in enumerate((b_xn, b_yn, b_dg)):
        r = pltpu.make_async_remote_copy(
            src_ref=comb_ref,
            dst_ref=out_ref.at[pl.ds(bsrc, 1)],
            send_sem=psend.at[i],
            recv_sem=precv.at[i],
            device_id=(x, y, z),
            device_id_type=pl.DeviceIdType.MESH,
        )
        recvs.append(r)
    for c in sends:
        c.wait_send()
    for r in recvs:
        r.wait_recv()


def _combine(o_part, m, l):
    return pl.pallas_call(
        _combine_body,
        in_specs=[
            pl.BlockSpec(memory_space=pltpu.VMEM),
            pl.BlockSpec(memory_space=pltpu.VMEM),
            pl.BlockSpec(memory_space=pltpu.VMEM),
        ],
        out_shape=jax.ShapeDtypeStruct((B, SQ, H, D), jnp.float32),
        out_specs=pl.BlockSpec(memory_space=pltpu.VMEM),
        scratch_shapes=[
            pltpu.VMEM((1, SQ, H, D), jnp.float32),
            pltpu.VMEM((1, H, SQ, 1), jnp.float32),
            pltpu.VMEM((1, H, SQ, 1), jnp.float32),
            pltpu.VMEM((1, SQ, H, D), jnp.float32),
            pltpu.SemaphoreType.DMA((3,)),
            pltpu.SemaphoreType.DMA((3,)),
            pltpu.SemaphoreType.DMA((3,)),
            pltpu.SemaphoreType.DMA((3,)),
        ],
        compiler_params=pltpu.CompilerParams(collective_id=0),
    )(o_part, m, l)


def kernel(Q, K, V):
    x = lax.axis_index("x")
    y = lax.axis_index("y")
    b_idx = jnp.full((1,), 2 * x + y, jnp.int32)
    o_part, m, l = _local_flash(b_idx, Q, K, V)
    return _combine(o_part, m, l)


# device time: 36441 ns/iter; 6.0630x vs baseline; 1.0015x over previous
import jax
import jax.numpy as jnp
from jax import lax
from jax.experimental import pallas as pl
from jax.experimental.pallas import tpu as pltpu

B, SQ, SKV, H, D = 4, 32, 4096, 8, 128
SCALE = D ** -0.5

BKV = 1024
NKV = SKV // BKV


def _body(
    b_ref,
    q_ref,
    k_ref,
    v_ref,
    out_ref,
    acc_o,
    acc_m,
    acc_l,
    kt,
    vt,
    op_ref,
    om_ref,
    ol_ref,
    ro_ref,
    rm_ref,
    rl_ref,
    comb_ref,
    zsend,
    zrecv,
    psend,
    precv,
):
    nk = pl.program_id(0)

    @pl.when(nk == 0)
    def _():
        acc_m[...] = jnp.full((H, SQ, 1), -jnp.inf, jnp.float32)
        acc_l[...] = jnp.zeros((H, SQ, 1), jnp.float32)
        acc_o[...] = jnp.zeros((SQ, H, D), jnp.float32)

    kt[...] = jnp.swapaxes(k_ref[0], 0, 1)
    vt[...] = jnp.swapaxes(v_ref[0], 0, 1)
    for h in range(H):
        q = q_ref[0, :, h, :]
        k = kt[h]
        v = vt[h]
        s = lax.dot_general(
            q, k, (((1,), (1,)), ((), ())), preferred_element_type=jnp.float32
        ) * SCALE
        m_prev = acc_m[h]
        m_blk = jnp.max(s, axis=1, keepdims=True)
        m_new = jnp.maximum(m_prev, m_blk)
        alpha = jnp.exp(m_prev - m_new)
        p = jnp.exp(s - m_new)
        l_new = acc_l[h] * alpha + jnp.sum(p, axis=1, keepdims=True)
        o_new = acc_o[:, h, :] * alpha + lax.dot_general(
            p, v, (((1,), (0,)), ((), ())), preferred_element_type=jnp.float32
        )
        acc_m[h] = m_new
        acc_l[h] = l_new
        acc_o[:, h, :] = o_new

    @pl.when(nk == NKV - 1)
    def _():
        x = lax.axis_index("x")
        y = lax.axis_index("y")
        z = lax.axis_index("z")
        my_b = 2 * x + y
        partner = (x, y, 1 - z)
        xn = (1 - x, y, z)
        yn = (x, 1 - y, z)
        dg = (1 - x, 1 - y, z)
        b_xn = 2 * (1 - x) + y
        b_yn = 2 * x + (1 - y)
        b_dg = 2 * (1 - x) + (1 - y)

        for h in range(H):
            op_ref[0, :, h, :] = acc_o[:, h, :] / acc_l[h]
        om_ref[0] = acc_m[...]
        ol_ref[0] = acc_l[...]

        barrier = pltpu.get_barrier_semaphore()
        for nbr in (partner, xn, yn, dg):
            pl.semaphore_signal(
                barrier,
                inc=1,
                device_id=nbr,
                device_id_type=pl.DeviceIdType.MESH,
            )
        pl.semaphore_wait(barrier, 4)

        zc = []
        for i, (src, dst) in enumerate(
            ((op_ref, ro_ref), (om_ref, rm_ref), (ol_ref, rl_ref))
        ):
            c = pltpu.make_async_remote_copy(
                src_ref=src,
                dst_ref=dst,
                send_sem=zsend.at[i],
                recv_sem=zrecv.at[i],
                device_id=partner,
                device_id_type=pl.DeviceIdType.MESH,
            )
            c.start()
            zc.append(c)
        for c in zc:
            c.wait()

        m_a = om_ref[...]
        m_b = rm_ref[...]
        m_n = jnp.maximum(m_a, m_b)
        a = jnp.exp(m_a - m_n) * ol_ref[...]
        bb = jnp.exp(m_b - m_n) * rl_ref[...]
        den = a + bb
        wa = a / den
        wb = bb / den
        for h in range(H):
            comb_ref[0, :, h, :] = (
                op_ref[0, :, h, :] * wa[0, h] + ro_ref[0, :, h, :] * wb[0, h]
            )
        out_ref[pl.ds(my_b, 1)] = comb_ref[...]

        sends = []
        for i, tgt in enumerate((xn, yn, dg)):
            c = pltpu.make_async_remote_copy(
                src_ref=comb_ref,
                dst_ref=out_ref.at[pl.ds(my_b, 1)],
                send_sem=psend.at[i],
                recv_sem=precv.at[i],
                device_id=tgt,
                device_id_type=pl.DeviceIdType.MESH,
            )
            c.start()
            sends.append(c)
        recvs = []
        for i, bsrc in enumerate((b_xn, b_yn, b_dg)):
            r = pltpu.make_async_remote_copy(
                src_ref=comb_ref,
                dst_ref=out_ref.at[pl.ds(bsrc, 1)],
                send_sem=psend.at[i],
                recv_sem=precv.at[i],
                device_id=(x, y, z),
                device_id_type=pl.DeviceIdType.MESH,
            )
            recvs.append(r)
        for c in sends:
            c.wait_send()
        for r in recvs:
            r.wait_recv()


def kernel(Q, K, V):
    x = lax.axis_index("x")
    y = lax.axis_index("y")
    b_idx = jnp.full((1,), 2 * x + y, jnp.int32)
    return pl.pallas_call(
        _body,
        grid_spec=pltpu.PrefetchScalarGridSpec(
            num_scalar_prefetch=1,
            grid=(NKV,),
            in_specs=[
                pl.BlockSpec((1, SQ, H, D), lambda nk, b: (b[0], 0, 0, 0)),
                pl.BlockSpec((1, BKV, H, D), lambda nk, b: (b[0], nk, 0, 0)),
                pl.BlockSpec((1, BKV, H, D), lambda nk, b: (b[0], nk, 0, 0)),
            ],
            out_specs=pl.BlockSpec(
                (B, SQ, H, D), lambda nk, b: (0, 0, 0, 0)
            ),
            scratch_shapes=[
                pltpu.VMEM((SQ, H, D), jnp.float32),
                pltpu.VMEM((H, SQ, 1), jnp.float32),
                pltpu.VMEM((H, SQ, 1), jnp.float32),
                pltpu.VMEM((H, BKV, D), jnp.float32),
                pltpu.VMEM((H, BKV, D), jnp.float32),
                pltpu.VMEM((1, SQ, H, D), jnp.float32),
                pltpu.VMEM((1, H, SQ, 1), jnp.float32),
                pltpu.VMEM((1, H, SQ, 1), jnp.float32),
                pltpu.VMEM((1, SQ, H, D), jnp.float32),
                pltpu.VMEM((1, H, SQ, 1), jnp.float32),
                pltpu.VMEM((1, H, SQ, 1), jnp.float32),
                pltpu.VMEM((1, SQ, H, D), jnp.float32),
                pltpu.SemaphoreType.DMA((3,)),
                pltpu.SemaphoreType.DMA((3,)),
                pltpu.SemaphoreType.DMA((3,)),
                pltpu.SemaphoreType.DMA((3,)),
            ],
        ),
        out_shape=jax.ShapeDtypeStruct((B, SQ, H, D), jnp.float32),
        compiler_params=pltpu.CompilerParams(collective_id=0),
    )(b_idx, Q, K, V)


# device time: 34642 ns/iter; 6.3779x vs baseline; 1.0519x over previous
import jax
import jax.numpy as jnp
from jax import lax
from jax.experimental import pallas as pl
from jax.experimental.pallas import tpu as pltpu

B, SQ, SKV, H, D = 4, 32, 4096, 8, 128
SCALE = D ** -0.5

BKV = 1024
NKV = SKV // BKV


def _body(
    b_ref,
    q_ref,
    k_ref,
    v_ref,
    out_ref,
    acc_o,
    acc_m,
    acc_l,
    kt,
    vt,
    op_ref,
    om_ref,
    ol_ref,
    ro_ref,
    rm_ref,
    rl_ref,
    comb_ref,
    agr_ref,
    zsend,
    zrecv,
    psend,
    precv,
):
    nk = pl.program_id(0)

    @pl.when(nk == 0)
    def _():
        acc_m[...] = jnp.full((H, SQ, 1), -jnp.inf, jnp.float32)
        acc_l[...] = jnp.zeros((H, SQ, 1), jnp.float32)
        acc_o[...] = jnp.zeros((SQ, H, D), jnp.float32)
        x = lax.axis_index("x")
        y = lax.axis_index("y")
        z = lax.axis_index("z")
        barrier = pltpu.get_barrier_semaphore()
        for nbr in ((x, y, 1 - z), (1 - x, y, z), (x, 1 - y, z), (1 - x, 1 - y, z)):
            pl.semaphore_signal(
                barrier,
                inc=1,
                device_id=nbr,
                device_id_type=pl.DeviceIdType.MESH,
            )
        pl.semaphore_wait(barrier, 4)

    kt[...] = jnp.swapaxes(k_ref[0], 0, 1)
    vt[...] = jnp.swapaxes(v_ref[0], 0, 1)
    for h in range(H):
        q = q_ref[0, :, h, :]
        k = kt[h]
        v = vt[h]
        s = lax.dot_general(
            q, k, (((1,), (1,)), ((), ())), preferred_element_type=jnp.float32
        ) * SCALE
        m_prev = acc_m[h]
        m_blk = jnp.max(s, axis=1, keepdims=True)
        m_new = jnp.maximum(m_prev, m_blk)
        alpha = jnp.exp(m_prev - m_new)
        p = jnp.exp(s - m_new)
        l_new = acc_l[h] * alpha + jnp.sum(p, axis=1, keepdims=True)
        o_new = acc_o[:, h, :] * alpha + lax.dot_general(
            p, v, (((1,), (0,)), ((), ())), preferred_element_type=jnp.float32
        )
        acc_m[h] = m_new
        acc_l[h] = l_new
        acc_o[:, h, :] = o_new

    @pl.when(nk == NKV - 1)
    def _():
        x = lax.axis_index("x")
        y = lax.axis_index("y")
        z = lax.axis_index("z")
        my_b = 2 * x + y
        partner = (x, y, 1 - z)
        xn = (1 - x, y, z)
        yn = (x, 1 - y, z)
        dg = (1 - x, 1 - y, z)
        b_xn = 2 * (1 - x) + y
        b_yn = 2 * x + (1 - y)
        b_dg = 2 * (1 - x) + (1 - y)

        for h in range(H):
            op_ref[0, :, h, :] = (acc_o[:, h, :] / acc_l[h]).astype(
                jnp.bfloat16
            )
        om_ref[0] = acc_m[...]
        ol_ref[0] = acc_l[...]

        zc = []
        for i, (src, dst) in enumerate(
            ((op_ref, ro_ref), (om_ref, rm_ref), (ol_ref, rl_ref))
        ):
            c = pltpu.make_async_remote_copy(
                src_ref=src,
                dst_ref=dst,
                send_sem=zsend.at[i],
                recv_sem=zrecv.at[i],
                device_id=partner,
                device_id_type=pl.DeviceIdType.MESH,
            )
            c.start()
            zc.append(c)
        for c in zc:
            c.wait()

        m_a = om_ref[...]
        m_b = rm_ref[...]
        m_n = jnp.maximum(m_a, m_b)
        a = jnp.exp(m_a - m_n) * ol_ref[...]
        bb = jnp.exp(m_b - m_n) * rl_ref[...]
        den = a + bb
        wa = a / den
        wb = bb / den
        for h in range(H):
            comb_ref[:, h, :] = (
                op_ref[0, :, h, :] * wa[0, h] + ro_ref[0, :, h, :] * wb[0, h]
            ).astype(jnp.bfloat16)
        out_ref[pl.ds(my_b, 1)] = comb_ref[...][None].astype(jnp.float32)

        sends = []
        for i, tgt in enumerate((xn, yn, dg)):
            c = pltpu.make_async_remote_copy(
                src_ref=comb_ref,
                dst_ref=agr_ref.at[i],
                send_sem=psend.at[i],
                recv_sem=precv.at[i],
                device_id=tgt,
                device_id_type=pl.DeviceIdType.MESH,
            )
            c.start()
            sends.append(c)
        recvs = []
        for i in range(3):
            r = pltpu.make_async_remote_copy(
                src_ref=comb_ref,
                dst_ref=agr_ref.at[i],
                send_sem=psend.at[i],
                recv_sem=precv.at[i],
                device_id=(x, y, z),
                device_id_type=pl.DeviceIdType.MESH,
            )
            recvs.append(r)
        for c in sends:
            c.wait_send()
        for i, bsrc in enumerate((b_xn, b_yn, b_dg)):
            recvs[i].wait_recv()
            out_ref[pl.ds(bsrc, 1)] = agr_ref[i][None].astype(jnp.float32)


def kernel(Q, K, V):
    x = lax.axis_index("x")
    y = lax.axis_index("y")
    b_idx = jnp.full((1,), 2 * x + y, jnp.int32)
    return pl.pallas_call(
        _body,
        grid_spec=pltpu.PrefetchScalarGridSpec(
            num_scalar_prefetch=1,
            grid=(NKV,),
            in_specs=[
                pl.BlockSpec((1, SQ, H, D), lambda nk, b: (b[0], 0, 0, 0)),
                pl.BlockSpec((1, BKV, H, D), lambda nk, b: (b[0], nk, 0, 0)),
                pl.BlockSpec((1, BKV, H, D), lambda nk, b: (b[0], nk, 0, 0)),
            ],
            out_specs=pl.BlockSpec(
                (B, SQ, H, D), lambda nk, b: (0, 0, 0, 0)
            ),
            scratch_shapes=[
                pltpu.VMEM((SQ, H, D), jnp.float32),
                pltpu.VMEM((H, SQ, 1), jnp.float32),
                pltpu.VMEM((H, SQ, 1), jnp.float32),
                pltpu.VMEM((H, BKV, D), jnp.float32),
                pltpu.VMEM((H, BKV, D), jnp.float32),
                pltpu.VMEM((1, SQ, H, D), jnp.bfloat16),
                pltpu.VMEM((1, H, SQ, 1), jnp.float32),
                pltpu.VMEM((1, H, SQ, 1), jnp.float32),
                pltpu.VMEM((1, SQ, H, D), jnp.bfloat16),
                pltpu.VMEM((1, H, SQ, 1), jnp.float32),
                pltpu.VMEM((1, H, SQ, 1), jnp.float32),
                pltpu.VMEM((SQ, H, D), jnp.bfloat16),
                pltpu.VMEM((3, SQ, H, D), jnp.bfloat16),
                pltpu.SemaphoreType.DMA((3,)),
                pltpu.SemaphoreType.DMA((3,)),
                pltpu.SemaphoreType.DMA((3,)),
                pltpu.SemaphoreType.DMA((3,)),
            ],
        ),
        out_shape=jax.ShapeDtypeStruct((B, SQ, H, D), jnp.float32),
        compiler_params=pltpu.CompilerParams(collective_id=0),
    )(b_idx, Q, K, V)
